# baseline (speedup 1.0000x reference)
import jax
import jax.numpy as jnp
from jax import lax
from jax.experimental import pallas as pl
from jax.experimental.pallas import tpu as pltpu
from jax.experimental.pallas import tpu_sc as plsc

_MAX_CTX = 32768
_SEQ = 2048
_NC = 2
_NS = 16
_NW = _NC * _NS
_CHUNK = _MAX_CTX // _NW
_ICHUNK = _SEQ // _NW
_L = 16


def _body(params_hbm, cache_hbm, idx_hbm, out_hbm,
          pbuf, cbuf, obuf, ibuf, sem_p, sem_c, sem_o, sem_i):
    wid = lax.axis_index("s") * _NC + lax.axis_index("c")
    base = wid * _CHUNK
    wbase = 2 * base
    iwbase = 2 * wid * _ICHUNK

    cp_p = pltpu.async_copy(params_hbm, pbuf, sem_p)
    cp_c = pltpu.async_copy(cache_hbm.at[pl.ds(wbase, 2 * _CHUNK)], cbuf, sem_c)
    cp_p.wait()
    cp_c.wait()

    sp_vec = pbuf[pl.ds(0, _L)]
    st_vec = pbuf[pl.ds(_L, _L)]
    lane = lax.broadcasted_iota(jnp.int32, (_L,), 0)
    half = lane >> 1
    odd = (lane & 1) == 1
    neg1 = jnp.full((_L,), -1, jnp.int32)
    zero = jnp.zeros((_L,), jnp.int32)

    for g in range(2 * _CHUNK // _L):
        slot = half + (base + 8 * g)
        d = (slot - st_vec) & (_MAX_CTX - 1)
        val = d + st_vec
        cache_w = cbuf[pl.ds(g * _L, _L)]
        out = jnp.where(d < _SEQ, jnp.where(odd, val >> 31, val),
                        jnp.where(slot < sp_vec, cache_w, neg1))
        obuf[pl.ds(g * _L, _L)] = out

    ibase = wid * _ICHUNK
    for g in range(2 * _ICHUNK // _L):
        j_vec = half + (ibase + 8 * g)
        idx_v = (j_vec + st_vec) & (_MAX_CTX - 1)
        ibuf[pl.ds(g * _L, _L)] = jnp.where(odd, zero, idx_v)

    cp_o = pltpu.async_copy(obuf, out_hbm.at[pl.ds(wbase, 2 * _CHUNK)], sem_o)
    cp_i = pltpu.async_copy(ibuf, idx_hbm.at[pl.ds(iwbase, 2 * _ICHUNK)], sem_i)
    cp_o.wait()
    cp_i.wait()


def kernel(input_pos, cache_positions, seq_len):
    start = input_pos[0].astype(jnp.int32)
    st = start + (jnp.asarray(seq_len).astype(jnp.int32) - _SEQ)
    params = jnp.concatenate(
        [jnp.broadcast_to(start, (_L,)), jnp.broadcast_to(st, (_L,))])
    cache_words = lax.bitcast_convert_type(
        cache_positions, jnp.int32).reshape(2 * _MAX_CTX)

    sc_call = pl.kernel(
        _body,
        out_type=(jax.ShapeDtypeStruct((2 * _SEQ,), jnp.int32),
                  jax.ShapeDtypeStruct((2 * _MAX_CTX,), jnp.int32)),
        mesh=plsc.VectorSubcoreMesh(core_axis_name="c", subcore_axis_name="s",
                                    num_cores=_NC, num_subcores=_NS),
        scratch_types=[
            pltpu.VMEM((2 * _L,), jnp.int32),
            pltpu.VMEM((2 * _CHUNK,), jnp.int32),
            pltpu.VMEM((2 * _CHUNK,), jnp.int32),
            pltpu.VMEM((2 * _ICHUNK,), jnp.int32),
            pltpu.SemaphoreType.DMA,
            pltpu.SemaphoreType.DMA,
            pltpu.SemaphoreType.DMA,
            pltpu.SemaphoreType.DMA,
        ],
    )
    idx_words, out_words = sc_call(params, cache_words)
    indices = lax.bitcast_convert_type(
        idx_words.reshape(_SEQ, 2), jnp.int64)
    new_cache = lax.bitcast_convert_type(
        out_words.reshape(_MAX_CTX, 2), jnp.int64)
    return indices, new_cache


# speedup vs baseline: 2.8601x; 2.8601x over previous
import jax
import jax.numpy as jnp
from jax import lax
from jax.experimental import pallas as pl
from jax.experimental.pallas import tpu as pltpu
from jax.experimental.pallas import tpu_sc as plsc

_MAX_CTX = 32768
_SEQ = 2048
_NC = 2
_NS = 16
_NW = _NC * _NS
_CHUNK = _MAX_CTX // _NW
_ICHUNK = _SEQ // _NW
_L = 16


def _body(params_hbm, cache_hbm, idx_hbm, out_hbm,
          pbuf, cbuf, obuf, ibuf, sem_p, sem_c, sem_o, sem_i):
    wid = lax.axis_index("s") * _NC + lax.axis_index("c")
    base = wid * _CHUNK
    ibase = wid * _ICHUNK

    cp_p = pltpu.async_copy(params_hbm, pbuf, sem_p)
    cp_c = pltpu.async_copy(cache_hbm.at[pl.ds(base, _CHUNK)], cbuf, sem_c)
    cp_p.wait()

    sp_vec = pbuf[pl.ds(0, _L)]
    st_vec = pbuf[pl.ds(_L, _L)]
    lane = lax.broadcasted_iota(jnp.int32, (_L,), 0)
    neg1 = jnp.full((_L,), -1, jnp.int32)

    for k in range(_ICHUNK // _L):
        j_vec = lane + (ibase + k * _L)
        ibuf[pl.ds(k * _L, _L)] = (j_vec + st_vec) & (_MAX_CTX - 1)
    cp_i = pltpu.async_copy(ibuf, idx_hbm.at[pl.ds(ibase, _ICHUNK)], sem_i)

    cp_c.wait()
    for k in range(_CHUNK // _L):
        i_vec = lane + (base + k * _L)
        d = (i_vec - st_vec) & (_MAX_CTX - 1)
        cache_v = cbuf[pl.ds(k * _L, _L)]
        out = jnp.where(d < _SEQ, d + st_vec,
                        jnp.where(i_vec < sp_vec, cache_v, neg1))
        obuf[pl.ds(k * _L, _L)] = out

    cp_o = pltpu.async_copy(obuf, out_hbm.at[pl.ds(base, _CHUNK)], sem_o)
    cp_o.wait()
    cp_i.wait()


def kernel(input_pos, cache_positions, seq_len):
    out_dtype = cache_positions.dtype
    start = input_pos[0].astype(jnp.int32)
    st = start + (jnp.asarray(seq_len).astype(jnp.int32) - _SEQ)
    params = jnp.concatenate(
        [jnp.broadcast_to(start, (_L,)), jnp.broadcast_to(st, (_L,))])
    cache32 = cache_positions.astype(jnp.int32)

    sc_call = pl.kernel(
        _body,
        out_type=(jax.ShapeDtypeStruct((_SEQ,), jnp.int32),
                  jax.ShapeDtypeStruct((_MAX_CTX,), jnp.int32)),
        mesh=plsc.VectorSubcoreMesh(core_axis_name="c", subcore_axis_name="s",
                                    num_cores=_NC, num_subcores=_NS),
        scratch_types=[
            pltpu.VMEM((2 * _L,), jnp.int32),
            pltpu.VMEM((_CHUNK,), jnp.int32),
            pltpu.VMEM((_CHUNK,), jnp.int32),
            pltpu.VMEM((_ICHUNK,), jnp.int32),
            pltpu.SemaphoreType.DMA,
            pltpu.SemaphoreType.DMA,
            pltpu.SemaphoreType.DMA,
            pltpu.SemaphoreType.DMA,
        ],
    )
    idx32, out32 = sc_call(params, cache32)
    return idx32.astype(out_dtype), out32.astype(out_dtype)
